# Initial kernel scaffold; baseline (speedup 1.0000x reference)
import jax
import jax.numpy as jnp
from jax.experimental import pallas as pl


def kernel(node_feature, W):
    raise NotImplementedError("write your pallas kernel here")


# speedup vs baseline: 5.8641x; 5.8641x over previous
import functools

import jax
import jax.numpy as jnp
from jax import lax
from jax.experimental import pallas as pl
from jax.experimental.pallas import tpu as pltpu
from jax.experimental.pallas import tpu_sc as plsc

B, N, F = 4096, 64, 9
VOCAB, EMB = 100, 128
R = B * N
NC, NS = 2, 16
NW = NC * NS
RW = R // NW
C = 128
NCHUNK = RW // C
NBLK = NW * NCHUNK


@functools.cache
def _build_kernel():
    mesh = plsc.VectorSubcoreMesh(core_axis_name="c", subcore_axis_name="s")

    @functools.partial(
        pl.kernel,
        out_type=jax.ShapeDtypeStruct((R, EMB), jnp.float32),
        mesh=mesh,
        scratch_types=[
            pltpu.VMEM((F, C), jnp.int32),
            pltpu.VMEM((C, EMB), jnp.float32),
            pltpu.VMEM((1, C), jnp.int32),
            pltpu.VMEM_SHARED((NS * C, EMB), jnp.float32),
            pltpu.SemaphoreType.DMA,
            pltpu.SemaphoreType.DMA,
        ],
    )
    def emb_kernel(idx_hbm, w_hbm, out_hbm, idxv, gbuf, idconst, acc, gsem, ssem):
        cid = lax.axis_index("c")
        sid = lax.axis_index("s")
        wid = sid * NC + cid
        lane = lax.broadcasted_iota(jnp.int32, (16,), 0)
        for i in range(C // 16):
            idconst[0, pl.ds(i * 16, 16)] = lane + (sid * C + i * 16)

        def chunk_body(ci, carry):
            blk = wid * NCHUNK + ci
            pltpu.async_copy(idx_hbm.at[blk], idxv, gsem).wait()
            for f in range(F):
                pltpu.async_copy(w_hbm.at[idxv.at[f]], gbuf, gsem).wait()
                pltpu.async_copy(gbuf, acc.at[idconst.at[0]], ssem, add=(f > 0)).wait()
            pltpu.async_copy(
                acc.at[pl.ds(sid * C, C), :],
                out_hbm.at[pl.ds(blk * C, C), :],
                ssem,
            ).wait()
            return carry

        lax.fori_loop(0, NCHUNK, chunk_body, 0)

    return emb_kernel


@jax.jit
def kernel(node_feature, W):
    idx = node_feature.astype(jnp.int32) + jnp.arange(F, dtype=jnp.int32) * VOCAB
    idx_t = idx.reshape(R, F).T
    idx_all = idx_t.reshape(F, NBLK, C).transpose(1, 0, 2)
    w2 = W.reshape(F * VOCAB, EMB)
    out = _build_kernel()(idx_all, w2)
    return out.reshape(B, N, EMB)


# speedup vs baseline: 8.3088x; 1.4169x over previous
import functools

import jax
import jax.numpy as jnp
from jax import lax
from jax.experimental import pallas as pl
from jax.experimental.pallas import tpu as pltpu
from jax.experimental.pallas import tpu_sc as plsc

B, N, F = 4096, 64, 9
VOCAB, EMB = 100, 128
R = B * N
NC, NS = 2, 16
NW = NC * NS
RW = R // NW
C = 128
NCHUNK = RW // C
NBLK = NW * NCHUNK


@functools.cache
def _build_kernel():
    mesh = plsc.VectorSubcoreMesh(core_axis_name="c", subcore_axis_name="s")

    @functools.partial(
        pl.kernel,
        out_type=jax.ShapeDtypeStruct((R, EMB), jnp.float32),
        mesh=mesh,
        scratch_types=[
            pltpu.VMEM((2, F, C), jnp.int32),
            pltpu.VMEM((3, C, EMB), jnp.float32),
            pltpu.VMEM((2, C), jnp.int32),
            pltpu.VMEM_SHARED((2 * NS * C, EMB), jnp.float32),
            pltpu.SemaphoreType.DMA,
            pltpu.SemaphoreType.DMA,
            pltpu.SemaphoreType.DMA,
            pltpu.SemaphoreType.DMA,
        ],
    )
    def emb_kernel(idx_hbm, w_hbm, out_hbm, idxv, gbuf, idconst, acc,
                   gsem, ssem, isem, osem):
        cid = lax.axis_index("c")
        sid = lax.axis_index("s")
        wid = sid * NC + cid
        lane = lax.broadcasted_iota(jnp.int32, (16,), 0)
        for b in range(2):
            for i in range(C // 16):
                idconst[b, pl.ds(i * 16, 16)] = lane + (b * NS * C + sid * C + i * 16)

        def fire_idx(ci):
            blk = jnp.minimum(wid * NCHUNK + ci, NBLK - 1)
            pltpu.async_copy(idx_hbm.at[blk], idxv.at[ci % 2], isem)

        def wait_idx(ci):
            pltpu.make_async_copy(idx_hbm.at[0], idxv.at[ci % 2], isem).wait()

        def fire_gather(bank, f):
            pltpu.async_copy(w_hbm.at[idxv.at[bank, f]], gbuf.at[f % 3], gsem)

        def wait_gather():
            pltpu.make_async_copy(w_hbm.at[idxv.at[0, 0]], gbuf.at[0], gsem).wait()

        def fire_scatter(bank, f):
            pltpu.async_copy(gbuf.at[f % 3], acc.at[idconst.at[bank]], ssem,
                             add=(f > 0))

        def wait_scatter():
            pltpu.make_async_copy(gbuf.at[0], acc.at[idconst.at[0]], ssem).wait()

        def out_slices(ci):
            blk = wid * NCHUNK + ci
            src = acc.at[pl.ds((ci % 2) * NS * C + sid * C, C), :]
            dst = out_hbm.at[pl.ds(blk * C, C), :]
            return src, dst

        fire_idx(0)

        def chunk_body(ci, carry):
            bank = ci % 2
            wait_idx(ci)
            @pl.when(ci >= 2)
            def _():
                src, dst = out_slices(ci - 2)
                pltpu.make_async_copy(src, dst, osem).wait()

            fire_gather(bank, 0)
            fire_gather(bank, 1)
            fire_gather(bank, 2)
            fire_idx(ci + 1)
            wait_gather()
            fire_scatter(bank, 0)
            wait_scatter()
            for f in range(1, F):
                if f + 2 < F:
                    if f >= 2:
                        wait_scatter()
                    fire_gather(bank, f + 2)
                wait_gather()
                fire_scatter(bank, f)
            wait_scatter()
            wait_scatter()
            wait_scatter()
            src, dst = out_slices(ci)
            pltpu.async_copy(src, dst, osem)
            return carry

        lax.fori_loop(0, NCHUNK, chunk_body, 0)

        for ci in (NCHUNK - 2, NCHUNK - 1):
            src, dst = out_slices(ci)
            pltpu.make_async_copy(src, dst, osem).wait()
        pltpu.make_async_copy(idx_hbm.at[0], idxv.at[NCHUNK % 2], isem).wait()

    return emb_kernel


@jax.jit
def kernel(node_feature, W):
    idx = node_feature.astype(jnp.int32) + jnp.arange(F, dtype=jnp.int32) * VOCAB
    idx_t = idx.reshape(R, F).T
    idx_all = idx_t.reshape(F, NBLK, C).transpose(1, 0, 2)
    w2 = W.reshape(F * VOCAB, EMB)
    out = _build_kernel()(idx_all, w2)
    return out.reshape(B, N, EMB)


# speedup vs baseline: 11.6667x; 1.4041x over previous
import functools

import jax
import jax.numpy as jnp
from jax import lax
from jax.experimental import pallas as pl
from jax.experimental.pallas import tpu as pltpu
from jax.experimental.pallas import tpu_sc as plsc

B, N, F = 4096, 64, 9
VOCAB, EMB = 100, 128
R = B * N
NC, NS = 2, 16
NW = NC * NS
RW = R // NW
C = 128
NCHUNK = RW // C
NBLK = NW * NCHUNK
NP = 4
NT = NP + 1


def _pair_body(w0_ref, w1_ref, out_ref):
    out_ref[0, 0] = w1_ref[0] + w0_ref[0, 0]


@functools.cache
def _build_pair_tables():
    return pl.pallas_call(
        _pair_body,
        grid=(NP, VOCAB),
        in_specs=[
            pl.BlockSpec((1, 1, 1, EMB), lambda p, i: (p, i, 0, 0)),
            pl.BlockSpec((1, VOCAB, EMB), lambda p, i: (p, 0, 0)),
        ],
        out_specs=pl.BlockSpec((1, 1, VOCAB, EMB), lambda p, i: (p, i, 0, 0)),
        out_shape=jax.ShapeDtypeStruct((NP, VOCAB, VOCAB, EMB), jnp.float32),
    )


@functools.cache
def _build_sc_kernel():
    mesh = plsc.VectorSubcoreMesh(core_axis_name="c", subcore_axis_name="s")

    @functools.partial(
        pl.kernel,
        out_type=jax.ShapeDtypeStruct((R, EMB), jnp.float32),
        mesh=mesh,
        scratch_types=[
            pltpu.VMEM((2, NT, C), jnp.int32),
            pltpu.VMEM((NT, C, EMB), jnp.float32),
            pltpu.VMEM((2, C), jnp.int32),
            pltpu.VMEM_SHARED((2 * NS * C, EMB), jnp.float32),
            pltpu.SemaphoreType.DMA,
            pltpu.SemaphoreType.DMA,
            pltpu.SemaphoreType.DMA,
            pltpu.SemaphoreType.DMA,
            pltpu.SemaphoreType.DMA,
            pltpu.SemaphoreType.DMA,
            pltpu.SemaphoreType.DMA,
            pltpu.SemaphoreType.DMA,
            pltpu.SemaphoreType.DMA,
            pltpu.SemaphoreType.DMA,
            pltpu.SemaphoreType.DMA,
            pltpu.SemaphoreType.DMA,
        ],
    )
    def emb_kernel(idx_hbm, pair_hbm, w8_hbm, out_hbm,
                   idxv, gbuf, idconst, acc,
                   isem, osem, g0, g1, g2, g3, g4, s0, s1, s2, s3, s4):
        gsems = (g0, g1, g2, g3, g4)
        ssems = (s0, s1, s2, s3, s4)
        cid = lax.axis_index("c")
        sid = lax.axis_index("s")
        wid = sid * NC + cid
        lane = lax.broadcasted_iota(jnp.int32, (16,), 0)
        for b in range(2):
            for i in range(C // 16):
                idconst[b, pl.ds(i * 16, 16)] = lane + (b * NS * C + sid * C + i * 16)

        def table_ref(bank, f):
            src = pair_hbm if f < NP else w8_hbm
            return src.at[idxv.at[bank, f]]

        def fire_idx(ci):
            blk = jnp.minimum(wid * NCHUNK + ci, NBLK - 1)
            pltpu.async_copy(idx_hbm.at[blk], idxv.at[ci % 2], isem)

        def acc_slice(bank):
            return acc.at[pl.ds(bank * NS * C + sid * C, C), :]

        def out_slices(ci):
            blk = wid * NCHUNK + ci
            return acc_slice(ci % 2), out_hbm.at[pl.ds(blk * C, C), :]

        fire_idx(0)

        def chunk_body(ci, carry):
            bank = ci % 2
            pltpu.make_async_copy(idx_hbm.at[0], idxv.at[bank], isem).wait()
            @pl.when(ci >= 2)
            def _():
                src, dst = out_slices(ci - 2)
                pltpu.make_async_copy(src, dst, osem).wait()

            for f in range(NT):
                pltpu.async_copy(table_ref(bank, f), gbuf.at[f], gsems[f])
            fire_idx(ci + 1)
            pltpu.make_async_copy(table_ref(bank, 0), gbuf.at[0], gsems[0]).wait()
            pltpu.async_copy(gbuf.at[0], acc.at[idconst.at[bank]], ssems[0])
            pltpu.make_async_copy(gbuf.at[0], acc.at[idconst.at[bank]],
                                  ssems[0]).wait()
            for f in range(1, NT):
                pltpu.make_async_copy(table_ref(bank, f), gbuf.at[f], gsems[f]).wait()
                pltpu.async_copy(gbuf.at[f], acc.at[idconst.at[bank]], ssems[f],
                                 add=True)
            for f in range(1, NT):
                pltpu.make_async_copy(gbuf.at[f], acc.at[idconst.at[bank]],
                                      ssems[f]).wait()
            src, dst = out_slices(ci)
            pltpu.async_copy(src, dst, osem)
            return carry

        lax.fori_loop(0, NCHUNK, chunk_body, 0)

        for ci in (NCHUNK - 2, NCHUNK - 1):
            src, dst = out_slices(ci)
            pltpu.make_async_copy(src, dst, osem).wait()
        pltpu.make_async_copy(idx_hbm.at[0], idxv.at[NCHUNK % 2], isem).wait()

    return emb_kernel


@jax.jit
def kernel(node_feature, W):
    idx = node_feature.astype(jnp.int32)
    pidx = (idx[..., 0:2 * NP:2] * VOCAB + idx[..., 1:2 * NP:2]
            + jnp.arange(NP, dtype=jnp.int32) * (VOCAB * VOCAB))
    allidx = jnp.concatenate([pidx, idx[..., 2 * NP:]], axis=-1)
    idx_all = (
        allidx.reshape(R, NT).T.reshape(NT, NBLK, C).transpose(1, 0, 2)
    )
    pairs = _build_pair_tables()(
        W[0:2 * NP:2].reshape(NP, VOCAB, 1, EMB), W[1:2 * NP:2]
    ).reshape(NP * VOCAB * VOCAB, EMB)
    out = _build_sc_kernel()(idx_all, pairs, W[2 * NP])
    return out.reshape(B, N, EMB)
